# Initial kernel scaffold; baseline (speedup 1.0000x reference)
import jax
import jax.numpy as jnp
from jax.experimental import pallas as pl


def kernel(logits):
    raise NotImplementedError("write your pallas kernel here")


# speedup vs baseline: 175.3700x; 175.3700x over previous
import functools

import jax
import jax.numpy as jnp
from jax.experimental import pallas as pl
from jax.experimental.pallas import tpu as pltpu

_TAU = 0.6
_EPS = 1e-06


def _phase1_kernel(x_ref, conf_ref, idx_ref, hist_ref):
    b = pl.program_id(0)
    h = pl.program_id(1)

    @pl.when(jnp.logical_and(b == 0, h == 0))
    def _init():
        hist_ref[...] = jnp.zeros_like(hist_ref)

    x = x_ref[0]
    C = x.shape[0]
    m = x[0]
    idx = jnp.zeros(m.shape, jnp.int32)
    for c in range(1, C):
        xc = x[c]
        gt = xc > m
        m = jnp.where(gt, xc, m)
        idx = jnp.where(gt, c, idx)
    s = jnp.exp(x[0] - m)
    for c in range(1, C):
        s = s + jnp.exp(x[c] - m)
    conf = 1.0 / s
    conf_ref[0] = conf
    idx_ref[0] = idx

    high = (conf > _TAU).astype(jnp.float32)
    cls = jax.lax.broadcasted_iota(jnp.int32, (C,) + idx.shape, 0)
    onehot = jnp.where(idx[None] == cls, high[None], 0.0)
    hist_ref[...] += jnp.sum(onehot, axis=(1, 2)).reshape(1, C)


def _phase2_kernel(sigma_ref, conf_ref, idx_ref, tc_ref, delta_ref):
    sigma = sigma_ref[0]
    C = sigma.shape[0]
    sigma_hat = sigma / jnp.maximum(jnp.max(sigma), _EPS)
    t_c = sigma_hat / (2.0 - jnp.minimum(sigma_hat, 1.0)) * _TAU

    @pl.when(pl.program_id(0) == 0)
    def _write_tc():
        tc_ref[...] = t_c.reshape(1, C)

    conf = conf_ref[...]
    idx = idx_ref[...]
    t_map = jnp.zeros_like(conf)
    for c in range(C):
        t_map = jnp.where(idx == c, t_c[c], t_map)
    delta_ref[...] = (conf > t_map).astype(jnp.uint8)


@jax.jit
def kernel(logits):
    B, C, H, W = logits.shape
    BH = 64

    conf, idx, sigma = pl.pallas_call(
        _phase1_kernel,
        grid=(B, H // BH),
        in_specs=[
            pl.BlockSpec((1, C, BH, W), lambda b, h: (b, 0, h, 0)),
        ],
        out_specs=[
            pl.BlockSpec((1, BH, W), lambda b, h: (b, h, 0)),
            pl.BlockSpec((1, BH, W), lambda b, h: (b, h, 0)),
            pl.BlockSpec((1, C), lambda b, h: (0, 0)),
        ],
        out_shape=[
            jax.ShapeDtypeStruct((B, H, W), jnp.float32),
            jax.ShapeDtypeStruct((B, H, W), jnp.int32),
            jax.ShapeDtypeStruct((1, C), jnp.float32),
        ],
    )(logits)

    R = B * H
    BR = min(512, R)
    conf2 = conf.reshape(R, W)
    idx2 = idx.reshape(R, W)

    t_c, delta = pl.pallas_call(
        _phase2_kernel,
        grid=(R // BR,),
        in_specs=[
            pl.BlockSpec((1, C), lambda r: (0, 0)),
            pl.BlockSpec((BR, W), lambda r: (r, 0)),
            pl.BlockSpec((BR, W), lambda r: (r, 0)),
        ],
        out_specs=[
            pl.BlockSpec((1, C), lambda r: (0, 0)),
            pl.BlockSpec((BR, W), lambda r: (r, 0)),
        ],
        out_shape=[
            jax.ShapeDtypeStruct((1, C), jnp.float32),
            jax.ShapeDtypeStruct((R, W), jnp.uint8),
        ],
    )(sigma, conf2, idx2)

    return delta.reshape(B, H, W).astype(jnp.bool_), t_c.reshape(C)
